# baseline (speedup 1.0000x reference)
import functools

import jax
import jax.numpy as jnp
from jax import lax
from jax.experimental import pallas as pl
from jax.experimental.pallas import tpu as pltpu
from jax.experimental.pallas import tpu_sc as plsc

VOCAB = 1000000
EMBED = 64
B = 4096
L = 50
N = B * L
NC = 2
NS = 16
NW = NC * NS
PER_W = N // NW
CHUNK = 640
NGROUP = CHUNK // 16
EPS = 1e-5


def _rsqrt(x):
    i = plsc.bitcast(x, jnp.int32)
    y = plsc.bitcast(jnp.int32(0x5F3759DF) - (i >> 1), jnp.float32)
    for _ in range(3):
        y = y * (1.5 - 0.5 * x * y * y)
    return y


def _sc_body(idx_hbm, wgt_hbm, table_hbm, gamma_hbm, beta_hbm, out_hbm,
             idx_v, wgt_v, rows_v, gam_v, bet_v, sem):
    c = lax.axis_index("c")
    s = lax.axis_index("s")
    wid = s * NC + c
    base = wid * PER_W

    pltpu.sync_copy(idx_hbm.at[pl.ds(base, PER_W)], idx_v)
    pltpu.sync_copy(wgt_hbm.at[pl.ds(base, PER_W)], wgt_v)
    pltpu.sync_copy(gamma_hbm, gam_v)
    pltpu.sync_copy(beta_hbm, bet_v)

    gam = [gam_v[pl.ds(16 * i, 16)] for i in range(4)]
    bet = [bet_v[pl.ds(16 * i, 16)] for i in range(4)]
    lane = lax.iota(jnp.int32, 16)

    def chunk_body(ci, carry):
        off = ci * CHUNK
        cp = pltpu.async_copy(table_hbm.at[idx_v.at[pl.ds(off, CHUNK)]],
                              rows_v, sem)
        cp.wait()

        def group_body(g, carry2):
            tbase = g * 16
            w = wgt_v[pl.ds(off + tbase, 16)]
            sums = jnp.zeros((16,), jnp.float32)
            sqs = jnp.zeros((16,), jnp.float32)
            for t in range(16):
                tok = tbase + t
                v = [rows_v[tok, pl.ds(16 * i, 16)] for i in range(4)]
                s_ = (v[0] + v[1]) + (v[2] + v[3])
                q_ = (v[0] * v[0] + v[1] * v[1]) + (v[2] * v[2] + v[3] * v[3])
                tm = lane == t
                sums = jnp.where(tm, jnp.sum(s_), sums)
                sqs = jnp.where(tm, jnp.sum(q_), sqs)
            mean_t = sums * (1.0 / 64.0)
            var_t = sqs * (1.0 / 64.0) - mean_t * mean_t
            var_x = var_t * w * w
            rstd = _rsqrt(var_x + EPS)
            a_vec = w * rstd
            m_vec = mean_t * w * rstd
            for t in range(16):
                tok = tbase + t
                tt = jnp.full((16,), t, jnp.int32)
                at = a_vec.at[tt].get(mode="promise_in_bounds")
                mt = m_vec.at[tt].get(mode="promise_in_bounds")
                for i in range(4):
                    vi = rows_v[tok, pl.ds(16 * i, 16)]
                    rows_v[tok, pl.ds(16 * i, 16)] = \
                        (vi * at - mt) * gam[i] + bet[i]
            return carry2

        lax.fori_loop(0, NGROUP, group_body, 0)
        pltpu.sync_copy(rows_v, out_hbm.at[pl.ds(base + off, CHUNK)])
        return carry

    lax.fori_loop(0, PER_W // CHUNK, chunk_body, 0)


@jax.jit
def _run(idx_flat, wgt_flat, table, gamma, beta):
    mesh = plsc.VectorSubcoreMesh(core_axis_name="c", subcore_axis_name="s")
    f = pl.kernel(
        _sc_body,
        out_type=jax.ShapeDtypeStruct((N, EMBED), jnp.float32),
        mesh=mesh,
        scratch_types=[
            pltpu.VMEM((PER_W,), jnp.int32),
            pltpu.VMEM((PER_W,), jnp.float32),
            pltpu.VMEM((CHUNK, EMBED), jnp.float32),
            pltpu.VMEM((EMBED,), jnp.float32),
            pltpu.VMEM((EMBED,), jnp.float32),
            pltpu.SemaphoreType.DMA,
        ],
        compiler_params=pltpu.CompilerParams(needs_layout_passes=False,
                                             use_tc_tiling_on_sc=False),
    )
    return f(idx_flat, wgt_flat, table, gamma, beta)


def kernel(idx, wgt, table, ln_gamma, ln_beta):
    idx_flat = idx.reshape(N).astype(jnp.int32)
    wgt_flat = wgt.reshape(N)
    out = _run(idx_flat, wgt_flat, table, ln_gamma, ln_beta)
    return out.reshape(B, L, EMBED)


# speedup vs baseline: 1.0822x; 1.0822x over previous
import functools

import jax
import jax.numpy as jnp
from jax import lax
from jax.experimental import pallas as pl
from jax.experimental.pallas import tpu as pltpu
from jax.experimental.pallas import tpu_sc as plsc

VOCAB = 1000000
EMBED = 64
B = 4096
L = 50
N = B * L
NC = 2
NS = 16
NW = NC * NS
PER_W = N // NW
CHUNK = 640
NGROUP = CHUNK // 16
EPS = 1e-5


def _rsqrt(x):
    i = plsc.bitcast(x, jnp.int32)
    y = plsc.bitcast(jnp.int32(0x5F3759DF) - (i >> 1), jnp.float32)
    for _ in range(3):
        y = y * (1.5 - 0.5 * x * y * y)
    return y


def _sc_body(idx_hbm, wgt_hbm, table_hbm, gamma_hbm, beta_hbm, out_hbm,
             idx_v, wgt_v, rows_v, gam_v, bet_v, sem):
    c = lax.axis_index("c")
    s = lax.axis_index("s")
    wid = s * NC + c
    base = wid * PER_W

    pltpu.sync_copy(idx_hbm.at[pl.ds(base, PER_W)], idx_v)
    pltpu.sync_copy(wgt_hbm.at[pl.ds(base, PER_W)], wgt_v)
    pltpu.sync_copy(gamma_hbm, gam_v)
    pltpu.sync_copy(beta_hbm, bet_v)

    gam = [gam_v[pl.ds(16 * i, 16)] for i in range(4)]
    bet = [bet_v[pl.ds(16 * i, 16)] for i in range(4)]
    lane = lax.iota(jnp.int32, 16)

    def chunk_body(ci, carry):
        off = ci * CHUNK
        cp = pltpu.async_copy(table_hbm.at[idx_v.at[pl.ds(off, CHUNK)]],
                              rows_v, sem)
        cp.wait()

        def group_body(g, carry2):
            tbase = g * 16
            w = wgt_v[pl.ds(off + tbase, 16)]
            sums = jnp.zeros((16,), jnp.float32)
            sqs = jnp.zeros((16,), jnp.float32)
            for t in range(16):
                tok = tbase + t
                v = [rows_v[tok, pl.ds(16 * i, 16)] for i in range(4)]
                s_ = (v[0] + v[1]) + (v[2] + v[3])
                q_ = (v[0] * v[0] + v[1] * v[1]) + (v[2] * v[2] + v[3] * v[3])
                tm = lane == t
                sums = jnp.where(tm, jnp.sum(s_), sums)
                sqs = jnp.where(tm, jnp.sum(q_), sqs)
            mean_t = sums * (1.0 / 64.0)
            var_t = sqs * (1.0 / 64.0) - mean_t * mean_t
            var_x = var_t * w * w
            rstd = _rsqrt(var_x + EPS)
            a_vec = w * rstd
            m_vec = mean_t * w * rstd
            for t in range(16):
                tok = tbase + t
                tt = jnp.full((16,), t, jnp.int32)
                at = a_vec.at[tt].get(mode="promise_in_bounds")
                mt = m_vec.at[tt].get(mode="promise_in_bounds")
                for i in range(4):
                    vi = rows_v[tok, pl.ds(16 * i, 16)]
                    rows_v[tok, pl.ds(16 * i, 16)] = \
                        (vi * at - mt) * gam[i] + bet[i]
            return carry2

        lax.fori_loop(0, NGROUP, group_body, 0)
        pltpu.sync_copy(rows_v, out_hbm.at[pl.ds(base + off, CHUNK)])
        return carry

    lax.fori_loop(0, PER_W // CHUNK, chunk_body, 0)


@jax.jit
def _run(idx_flat, wgt_flat, table, gamma, beta):
    mesh = plsc.VectorSubcoreMesh(core_axis_name="c", subcore_axis_name="s")
    f = pl.kernel(
        _sc_body,
        out_type=jax.ShapeDtypeStruct((N, EMBED), jnp.float32),
        mesh=mesh,
        scratch_types=[
            pltpu.VMEM((PER_W,), jnp.int32),
            pltpu.VMEM((PER_W,), jnp.float32),
            pltpu.VMEM((CHUNK, EMBED), jnp.float32),
            pltpu.VMEM((EMBED,), jnp.float32),
            pltpu.VMEM((EMBED,), jnp.float32),
            pltpu.SemaphoreType.DMA,
        ],
        compiler_params=pltpu.CompilerParams(needs_layout_passes=False,
                                             use_tc_tiling_on_sc=False),
    )
    return f(idx_flat, wgt_flat, table, gamma, beta)


def kernel(idx, wgt, table, ln_gamma, ln_beta):
    idx_flat = idx.reshape(N).astype(jnp.int32) * 2
    wgt_flat = wgt.reshape(N)
    table2 = jnp.pad(table, ((0, 0), (0, EMBED))).reshape(2 * VOCAB, EMBED)
    out = _run(idx_flat, wgt_flat, table2, ln_gamma, ln_beta)
    return out.reshape(B, L, EMBED)
